# Initial kernel scaffold; baseline (speedup 1.0000x reference)
import jax
import jax.numpy as jnp
from jax.experimental import pallas as pl


def kernel(x_ts, x_lc, x_ass, batch_ts, batch_lc, batch_ass, ts_w1, ts_b1, ts_w2, ts_b2, lc_w1, lc_b1, lc_w2, lc_b2, conv_w, conv_b, out_w1, out_b1, out_w2, out_b2, out_w3, out_b3, out_w4, out_b4):
    raise NotImplementedError("write your pallas kernel here")


# speedup vs baseline: 9.6444x; 9.6444x over previous
import functools

import jax
import jax.numpy as jnp
from jax.experimental import pallas as pl
from jax.experimental.pallas import tpu as pltpu

F = 128
AUG = 256
TILE = 256
CHUNK = 256
K = 24
NEG_INF = float("-inf")


def _elu(x):
    return jnp.where(x > 0, x, jnp.exp(jnp.minimum(x, 0.0)) - 1.0)


def _encoder_body(x_ref, w1_ref, b1_ref, w2_ref, b2_ref, o_ref):
    h = jax.lax.dot_general(x_ref[...], w1_ref[...], (((1,), (1,)), ((), ())),
                            preferred_element_type=jnp.float32)
    h = _elu(h + b1_ref[...])
    h = jax.lax.dot_general(h, w2_ref[...], (((1,), (1,)), ((), ())),
                            preferred_element_type=jnp.float32)
    o_ref[...] = _elu(h + b2_ref[...])


def _encode(x, w1, b1, w2, b2):
    n, fin = x.shape
    rt = 512
    return pl.pallas_call(
        _encoder_body,
        grid=(n // rt,),
        in_specs=[
            pl.BlockSpec((rt, fin), lambda i: (i, 0)),
            pl.BlockSpec(w1.shape, lambda i: (0, 0)),
            pl.BlockSpec((1, F), lambda i: (0, 0)),
            pl.BlockSpec((F, F), lambda i: (0, 0)),
            pl.BlockSpec((1, F), lambda i: (0, 0)),
        ],
        out_specs=pl.BlockSpec((rt, F), lambda i: (i, 0)),
        out_shape=jax.ShapeDtypeStruct((n, F), jnp.float32),
    )(x, w1, b1.reshape(1, F), w2, b2.reshape(1, F))


def _prep_body(x_ref, wu_ref, wv_ref, cb_ref, u_ref, v_ref, s_ref):
    x = x_ref[...]
    u_ref[...] = jax.lax.dot_general(x, wu_ref[...], (((1,), (1,)), ((), ())),
                                     preferred_element_type=jnp.float32) + cb_ref[...]
    v_ref[...] = jax.lax.dot_general(x, wv_ref[...], (((1,), (1,)), ((), ())),
                                     preferred_element_type=jnp.float32)
    sq = jnp.sum(x * x, axis=1, keepdims=True)
    lane = jax.lax.broadcasted_iota(jnp.int32, (x.shape[0], AUG - F), 1)
    s_ref[:, :F] = x
    s_ref[:, F:] = jnp.where(lane == 0, sq, 0.0)


def _prep(x, wu, wv, cb):
    n = x.shape[0]
    rt = 512
    return pl.pallas_call(
        _prep_body,
        grid=(n // rt,),
        in_specs=[
            pl.BlockSpec((rt, F), lambda i: (i, 0)),
            pl.BlockSpec((F, F), lambda i: (0, 0)),
            pl.BlockSpec((F, F), lambda i: (0, 0)),
            pl.BlockSpec((1, F), lambda i: (0, 0)),
        ],
        out_specs=[
            pl.BlockSpec((rt, F), lambda i: (i, 0)),
            pl.BlockSpec((rt, F), lambda i: (i, 0)),
            pl.BlockSpec((rt, AUG), lambda i: (i, 0)),
        ],
        out_shape=[
            jax.ShapeDtypeStruct((n, F), jnp.float32),
            jax.ShapeDtypeStruct((n, F), jnp.float32),
            jax.ShapeDtypeStruct((n, AUG), jnp.float32),
        ],
    )(x, wu, wv, cb.reshape(1, F))


def _conv_body(clo_ref, cn_ref, sd_ref, u_ref, bd_ref, s_ref, v_ref, bs_ref,
               o_ref, dw_ref, a_ref, vmax_ref, nsrc_chunks):
    t = pl.program_id(0)
    clo = clo_ref[t]
    cn = cn_ref[t]

    lane = jax.lax.broadcasted_iota(jnp.int32, (TILE, AUG), 1)
    sd = sd_ref[...]
    a_ref[...] = jnp.where(lane < F, -2.0 * sd,
                           jnp.where(lane == F, 1.0, 0.0))
    vmax_ref[...] = jnp.full((TILE, F), NEG_INF, jnp.float32)

    bd = bd_ref[...]

    def fill(ql, _):
        q = clo + ql
        sc = s_ref[pl.ds(q * CHUNK, CHUNK), :]
        d2 = jax.lax.dot_general(a_ref[...], sc, (((1,), (1,)), ((), ())),
                                 preferred_element_type=jnp.float32)
        bs = bs_ref[pl.ds(q, 1), :]
        dw_ref[:, pl.ds(ql * CHUNK, CHUNK)] = jnp.where(bd == bs, d2, jnp.inf)
        return 0

    jax.lax.fori_loop(0, cn, fill, 0)

    ciota = jax.lax.broadcasted_iota(jnp.int32, (TILE, CHUNK), 1)

    def round_body(r, _):
        def amin_chunk(ql, carry):
            mv, mi = carry
            w = dw_ref[:, pl.ds(ql * CHUNK, CHUNK)]
            cmin = jnp.min(w, axis=1, keepdims=True)
            lidx = jnp.where(w == cmin, ciota, CHUNK)
            cidx = jnp.min(lidx, axis=1, keepdims=True) + ql * CHUNK
            better = cmin < mv
            return (jnp.where(better, cmin, mv),
                    jnp.where(better, cidx, mi))

        mv0 = jnp.full((TILE, 1), jnp.inf, jnp.float32)
        mi0 = jnp.full((TILE, 1), jnp.int32(-1))
        mv, mi = jax.lax.fori_loop(0, cn, amin_chunk, (mv0, mi0))
        active = mv < jnp.inf

        def sel_chunk(ql, acc):
            base = ql * CHUNK
            onehot = (ciota == (mi - base)) & active
            w = dw_ref[:, pl.ds(base, CHUNK)]
            dw_ref[:, pl.ds(base, CHUNK)] = jnp.where(onehot, jnp.inf, w)
            vchunk = v_ref[pl.ds((clo + ql) * CHUNK, CHUNK), :]
            return acc + jax.lax.dot_general(
                onehot.astype(jnp.float32), vchunk, (((1,), (0,)), ((), ())),
                preferred_element_type=jnp.float32)

        vsel = jax.lax.fori_loop(0, cn, sel_chunk,
                                 jnp.zeros((TILE, F), jnp.float32))
        vmax_ref[...] = jnp.where(active,
                                  jnp.maximum(vmax_ref[...], vsel),
                                  vmax_ref[...])
        return 0

    jax.lax.fori_loop(0, K, round_body, 0)

    vmax = vmax_ref[...]
    res = u_ref[...] + vmax
    o_ref[...] = jnp.where(vmax == NEG_INF, NEG_INF, _elu(res))


def _edge_conv(u_dst, s_dst, bd_col, s_src, v_src, bs_row, clo, cn):
    n_dst = u_dst.shape[0]
    n_src = v_src.shape[0]
    nt = n_dst // TILE
    nsrc_chunks = n_src // CHUNK
    grid_spec = pltpu.PrefetchScalarGridSpec(
        num_scalar_prefetch=2,
        grid=(nt,),
        in_specs=[
            pl.BlockSpec((TILE, AUG), lambda i, *_: (i, 0)),
            pl.BlockSpec((TILE, F), lambda i, *_: (i, 0)),
            pl.BlockSpec((TILE, 1), lambda i, *_: (i, 0)),
            pl.BlockSpec((n_src, AUG), lambda i, *_: (0, 0)),
            pl.BlockSpec((n_src, F), lambda i, *_: (0, 0)),
            pl.BlockSpec((nsrc_chunks, CHUNK), lambda i, *_: (0, 0)),
        ],
        out_specs=pl.BlockSpec((TILE, F), lambda i, *_: (i, 0)),
        scratch_shapes=[
            pltpu.VMEM((TILE, n_src), jnp.float32),
            pltpu.VMEM((TILE, AUG), jnp.float32),
            pltpu.VMEM((TILE, F), jnp.float32),
        ],
    )
    body = functools.partial(_conv_body, nsrc_chunks=nsrc_chunks)
    return pl.pallas_call(
        body,
        grid_spec=grid_spec,
        out_shape=jax.ShapeDtypeStruct((n_dst, F), jnp.float32),
    )(clo, cn, s_dst, u_dst, bd_col, s_src, v_src, bs_row)


def _pool_body(x_ref, bl_ref, w1_ref, b1_ref, w2_ref, b2_ref, w3_ref, b3_ref,
               w4_ref, b4_ref, o_ref, sums_ref, cnt_ref, nb, nt):
    t = pl.program_id(0)

    @pl.when(t == 0)
    def _():
        sums_ref[...] = jnp.zeros_like(sums_ref)
        cnt_ref[...] = jnp.zeros_like(cnt_ref)

    bl = bl_ref[pl.ds(t, 1), :]
    gi = jax.lax.broadcasted_iota(jnp.int32, (nb, TILE), 0)
    oh = (gi == bl.astype(jnp.int32)).astype(jnp.float32)
    sums_ref[...] += jax.lax.dot_general(oh, x_ref[...], (((1,), (0,)), ((), ())),
                                         preferred_element_type=jnp.float32)
    cnt_ref[...] += jnp.sum(oh, axis=1, keepdims=True)

    @pl.when(t == nt - 1)
    def _():
        pooled = sums_ref[...] / jnp.maximum(cnt_ref[...], 1.0)

        def lin(h, w_ref, b_ref):
            return jax.lax.dot_general(h, w_ref[...], (((1,), (1,)), ((), ())),
                                       preferred_element_type=jnp.float32) + b_ref[...]

        h = _elu(lin(pooled, w1_ref, b1_ref))
        h = _elu(lin(h, w2_ref, b2_ref))
        h = _elu(lin(h, w3_ref, b3_ref))
        o_ref[...] = lin(h, w4_ref, b4_ref)


def _pool_head(x, bl_row, nb, w1, b1, w2, b2, w3, b3, w4, b4):
    n = x.shape[0]
    nt = n // TILE
    body = functools.partial(_pool_body, nb=nb, nt=nt)
    return pl.pallas_call(
        body,
        grid=(nt,),
        in_specs=[
            pl.BlockSpec((TILE, F), lambda i: (i, 0)),
            pl.BlockSpec(bl_row.shape, lambda i: (0, 0)),
            pl.BlockSpec(w1.shape, lambda i: (0, 0)),
            pl.BlockSpec((1, w1.shape[0]), lambda i: (0, 0)),
            pl.BlockSpec(w2.shape, lambda i: (0, 0)),
            pl.BlockSpec((1, w2.shape[0]), lambda i: (0, 0)),
            pl.BlockSpec(w3.shape, lambda i: (0, 0)),
            pl.BlockSpec((1, w3.shape[0]), lambda i: (0, 0)),
            pl.BlockSpec(w4.shape, lambda i: (0, 0)),
            pl.BlockSpec((1, w4.shape[0]), lambda i: (0, 0)),
        ],
        out_specs=pl.BlockSpec((nb, w4.shape[0]), lambda i: (0, 0)),
        out_shape=jax.ShapeDtypeStruct((nb, w4.shape[0]), jnp.float32),
        scratch_shapes=[
            pltpu.VMEM((nb, F), jnp.float32),
            pltpu.VMEM((nb, 1), jnp.float32),
        ],
    )(x, bl_row, w1, b1.reshape(1, -1), w2, b2.reshape(1, -1),
      w3, b3.reshape(1, -1), w4, b4.reshape(1, -1))


def _chunk_info(bd, bs, nb):
    g = jnp.arange(nb, dtype=bs.dtype)
    starts = jnp.searchsorted(bs, g, side="left").astype(jnp.int32)
    ends = jnp.searchsorted(bs, g, side="right").astype(jnp.int32)
    g_lo = bd[::TILE]
    g_hi = bd[TILE - 1::TILE]
    s_lo = starts[g_lo]
    s_hi = ends[g_hi]
    clo = s_lo // CHUNK
    cn = -(-s_hi // CHUNK) - clo
    return clo.astype(jnp.int32), jnp.maximum(cn, 0).astype(jnp.int32)


def kernel(x_ts, x_lc, x_ass, batch_ts, batch_lc, batch_ass,
           ts_w1, ts_b1, ts_w2, ts_b2, lc_w1, lc_b1, lc_w2, lc_b2,
           conv_w, conv_b, out_w1, out_b1, out_w2, out_b2,
           out_w3, out_b3, out_w4, out_b4):
    nb = 32
    n_ts = x_ts.shape[0]
    n_lc = x_lc.shape[0]

    w1c = conv_w[:, :F]
    w2c = conv_w[:, F:]
    wu = w1c - w2c
    wv = w2c

    x_ts_p = jnp.pad(x_ts, ((0, 0), (0, 2)))
    ts_w1_p = jnp.pad(ts_w1, ((0, 0), (0, 2)))
    bts_col = batch_ts.astype(jnp.float32).reshape(n_ts, 1)
    blc_col = batch_lc.astype(jnp.float32).reshape(n_lc, 1)
    bts_row = batch_ts.astype(jnp.float32).reshape(n_ts // CHUNK, CHUNK)
    blc_row = batch_lc.astype(jnp.float32).reshape(n_lc // CHUNK, CHUNK)
    lc_lo, lc_n = _chunk_info(batch_lc, batch_lc, nb)
    ts_lo, ts_n = _chunk_info(batch_ts, batch_ts, nb)
    x_lo, x_n = _chunk_info(batch_lc, batch_ts, nb)

    enc_ts = _encode(x_ts_p, ts_w1_p, ts_b1, ts_w2, ts_b2)
    enc_lc = _encode(x_lc, lc_w1, lc_b1, lc_w2, lc_b2)

    def conv_same(x, b_col, b_row, clo, cn):
        u, v, s = _prep(x, wu, wv, conv_b)
        return _edge_conv(u, s, b_col, s, v, b_row, clo, cn)

    feats1 = conv_same(enc_lc, blc_col, blc_row, lc_lo, lc_n)
    feats11 = conv_same(feats1, blc_col, blc_row, lc_lo, lc_n)
    feats2 = conv_same(enc_ts, bts_col, bts_row, ts_lo, ts_n)
    feats22 = conv_same(feats2, bts_col, bts_row, ts_lo, ts_n)

    u_d, _, s_d = _prep(feats11, wu, wv, conv_b)
    _, v_s, s_s = _prep(feats22, wu, wv, conv_b)
    feats3 = _edge_conv(u_d, s_d, blc_col, s_s, v_s, bts_row, x_lo, x_n)

    out = _pool_head(feats3, blc_row, nb, out_w1, out_b1, out_w2, out_b2,
                     out_w3, out_b3, out_w4, out_b4)
    batch_out = jnp.arange(nb, dtype=jnp.int32)
    return (out, batch_out)


# speedup vs baseline: 51.1019x; 5.2986x over previous
import functools

import jax
import jax.numpy as jnp
from jax.experimental import pallas as pl
from jax.experimental.pallas import tpu as pltpu

F = 128
AUG = 256
TILE = 256
CHUNK = 256
K = 24
NEG_INF = float("-inf")


def _elu(x):
    return jnp.where(x > 0, x, jnp.exp(jnp.minimum(x, 0.0)) - 1.0)


def _encoder_body(x_ref, w1_ref, b1_ref, w2_ref, b2_ref, o_ref):
    h = jax.lax.dot_general(x_ref[...], w1_ref[...], (((1,), (1,)), ((), ())),
                            preferred_element_type=jnp.float32)
    h = _elu(h + b1_ref[...])
    h = jax.lax.dot_general(h, w2_ref[...], (((1,), (1,)), ((), ())),
                            preferred_element_type=jnp.float32)
    o_ref[...] = _elu(h + b2_ref[...])


def _encode(x, w1, b1, w2, b2):
    n, fin = x.shape
    rt = 512
    return pl.pallas_call(
        _encoder_body,
        grid=(n // rt,),
        in_specs=[
            pl.BlockSpec((rt, fin), lambda i: (i, 0)),
            pl.BlockSpec(w1.shape, lambda i: (0, 0)),
            pl.BlockSpec((1, F), lambda i: (0, 0)),
            pl.BlockSpec((F, F), lambda i: (0, 0)),
            pl.BlockSpec((1, F), lambda i: (0, 0)),
        ],
        out_specs=pl.BlockSpec((rt, F), lambda i: (i, 0)),
        out_shape=jax.ShapeDtypeStruct((n, F), jnp.float32),
    )(x, w1, b1.reshape(1, F), w2, b2.reshape(1, F))


def _prep_body(x_ref, wu_ref, wv_ref, cb_ref, u_ref, v_ref, s_ref):
    x = x_ref[...]
    u_ref[...] = jax.lax.dot_general(x, wu_ref[...], (((1,), (1,)), ((), ())),
                                     preferred_element_type=jnp.float32) + cb_ref[...]
    v_ref[...] = jax.lax.dot_general(x, wv_ref[...], (((1,), (1,)), ((), ())),
                                     preferred_element_type=jnp.float32)
    sq = jnp.sum(x * x, axis=1, keepdims=True)
    lane = jax.lax.broadcasted_iota(jnp.int32, (x.shape[0], AUG - F), 1)
    s_ref[:, :F] = x
    s_ref[:, F:] = jnp.where(lane == 0, sq, 0.0)


def _prep(x, wu, wv, cb):
    n = x.shape[0]
    rt = 512
    return pl.pallas_call(
        _prep_body,
        grid=(n // rt,),
        in_specs=[
            pl.BlockSpec((rt, F), lambda i: (i, 0)),
            pl.BlockSpec((F, F), lambda i: (0, 0)),
            pl.BlockSpec((F, F), lambda i: (0, 0)),
            pl.BlockSpec((1, F), lambda i: (0, 0)),
        ],
        out_specs=[
            pl.BlockSpec((rt, F), lambda i: (i, 0)),
            pl.BlockSpec((rt, F), lambda i: (i, 0)),
            pl.BlockSpec((rt, AUG), lambda i: (i, 0)),
        ],
        out_shape=[
            jax.ShapeDtypeStruct((n, F), jnp.float32),
            jax.ShapeDtypeStruct((n, F), jnp.float32),
            jax.ShapeDtypeStruct((n, AUG), jnp.float32),
        ],
    )(x, wu, wv, cb.reshape(1, F))


def _conv_body(clo_ref, cn_ref, sd_ref, u_ref, bd_ref, s_ref, v_ref, bs_ref,
               o_ref, dw_ref, a_ref, vmax_ref, nsrc_chunks):
    t = pl.program_id(0)
    clo = clo_ref[t]
    cn = cn_ref[t]

    lane = jax.lax.broadcasted_iota(jnp.int32, (TILE, AUG), 1)
    sd = sd_ref[...]
    a_ref[...] = jnp.where(lane < F, -2.0 * sd,
                           jnp.where(lane == F, 1.0, 0.0))
    vmax_ref[...] = jnp.full((TILE, F), NEG_INF, jnp.float32)

    bd = bd_ref[...]

    def fill(ql, _):
        q = clo + ql
        sc = s_ref[pl.ds(q * CHUNK, CHUNK), :]
        d2 = jax.lax.dot_general(a_ref[...], sc, (((1,), (1,)), ((), ())),
                                 preferred_element_type=jnp.float32)
        bs = bs_ref[pl.ds(q, 1), :]
        dw_ref[:, pl.ds(ql * CHUNK, CHUNK)] = jnp.where(bd == bs, d2, jnp.inf)
        return 0

    jax.lax.fori_loop(0, cn, fill, 0)

    ciota = jax.lax.broadcasted_iota(jnp.int32, (TILE, CHUNK), 1)

    def round_body(r, _):
        def amin_chunk(ql, carry):
            mv, mi = carry
            w = dw_ref[:, pl.ds(ql * CHUNK, CHUNK)]
            cmin = jnp.min(w, axis=1, keepdims=True)
            lidx = jnp.where(w == cmin, ciota, CHUNK)
            cidx = jnp.min(lidx, axis=1, keepdims=True) + ql * CHUNK
            better = cmin < mv
            return (jnp.where(better, cmin, mv),
                    jnp.where(better, cidx, mi))

        mv0 = jnp.full((TILE, 1), jnp.inf, jnp.float32)
        mi0 = jnp.full((TILE, 1), jnp.int32(-1))
        mv, mi = jax.lax.fori_loop(0, cn, amin_chunk, (mv0, mi0))
        active = mv < jnp.inf

        def sel_chunk(ql, acc):
            base = ql * CHUNK
            onehot = (ciota == (mi - base)) & active
            w = dw_ref[:, pl.ds(base, CHUNK)]
            dw_ref[:, pl.ds(base, CHUNK)] = jnp.where(onehot, jnp.inf, w)
            vchunk = v_ref[pl.ds((clo + ql) * CHUNK, CHUNK), :]
            return acc + jax.lax.dot_general(
                onehot.astype(jnp.float32), vchunk, (((1,), (0,)), ((), ())),
                preferred_element_type=jnp.float32)

        vsel = jax.lax.fori_loop(0, cn, sel_chunk,
                                 jnp.zeros((TILE, F), jnp.float32))
        vmax_ref[...] = jnp.where(active,
                                  jnp.maximum(vmax_ref[...], vsel),
                                  vmax_ref[...])
        return 0

    jax.lax.fori_loop(0, K, round_body, 0, unroll=2)

    vmax = vmax_ref[...]
    res = u_ref[...] + vmax
    o_ref[...] = jnp.where(vmax == NEG_INF, NEG_INF, _elu(res))


def _edge_conv(u_dst, s_dst, bd_col, s_src, v_src, bs_row, clo, cn):
    n_dst = u_dst.shape[0]
    n_src = v_src.shape[0]
    nt = n_dst // TILE
    nsrc_chunks = n_src // CHUNK
    grid_spec = pltpu.PrefetchScalarGridSpec(
        num_scalar_prefetch=2,
        grid=(nt,),
        in_specs=[
            pl.BlockSpec((TILE, AUG), lambda i, *_: (i, 0)),
            pl.BlockSpec((TILE, F), lambda i, *_: (i, 0)),
            pl.BlockSpec((TILE, 1), lambda i, *_: (i, 0)),
            pl.BlockSpec((n_src, AUG), lambda i, *_: (0, 0)),
            pl.BlockSpec((n_src, F), lambda i, *_: (0, 0)),
            pl.BlockSpec((nsrc_chunks, CHUNK), lambda i, *_: (0, 0)),
        ],
        out_specs=pl.BlockSpec((TILE, F), lambda i, *_: (i, 0)),
        scratch_shapes=[
            pltpu.VMEM((TILE, n_src), jnp.float32),
            pltpu.VMEM((TILE, AUG), jnp.float32),
            pltpu.VMEM((TILE, F), jnp.float32),
        ],
    )
    body = functools.partial(_conv_body, nsrc_chunks=nsrc_chunks)
    return pl.pallas_call(
        body,
        grid_spec=grid_spec,
        out_shape=jax.ShapeDtypeStruct((n_dst, F), jnp.float32),
    )(clo, cn, s_dst, u_dst, bd_col, s_src, v_src, bs_row)


def _pool_body(x_ref, bl_ref, w1_ref, b1_ref, w2_ref, b2_ref, w3_ref, b3_ref,
               w4_ref, b4_ref, o_ref, sums_ref, cnt_ref, nb, nt):
    t = pl.program_id(0)

    @pl.when(t == 0)
    def _():
        sums_ref[...] = jnp.zeros_like(sums_ref)
        cnt_ref[...] = jnp.zeros_like(cnt_ref)

    bl = bl_ref[pl.ds(t, 1), :]
    gi = jax.lax.broadcasted_iota(jnp.int32, (nb, TILE), 0)
    oh = (gi == bl.astype(jnp.int32)).astype(jnp.float32)
    sums_ref[...] += jax.lax.dot_general(oh, x_ref[...], (((1,), (0,)), ((), ())),
                                         preferred_element_type=jnp.float32)
    cnt_ref[...] += jnp.sum(oh, axis=1, keepdims=True)

    @pl.when(t == nt - 1)
    def _():
        pooled = sums_ref[...] / jnp.maximum(cnt_ref[...], 1.0)

        def lin(h, w_ref, b_ref):
            return jax.lax.dot_general(h, w_ref[...], (((1,), (1,)), ((), ())),
                                       preferred_element_type=jnp.float32) + b_ref[...]

        h = _elu(lin(pooled, w1_ref, b1_ref))
        h = _elu(lin(h, w2_ref, b2_ref))
        h = _elu(lin(h, w3_ref, b3_ref))
        o_ref[...] = lin(h, w4_ref, b4_ref)


def _pool_head(x, bl_row, nb, w1, b1, w2, b2, w3, b3, w4, b4):
    n = x.shape[0]
    nt = n // TILE
    body = functools.partial(_pool_body, nb=nb, nt=nt)
    return pl.pallas_call(
        body,
        grid=(nt,),
        in_specs=[
            pl.BlockSpec((TILE, F), lambda i: (i, 0)),
            pl.BlockSpec(bl_row.shape, lambda i: (0, 0)),
            pl.BlockSpec(w1.shape, lambda i: (0, 0)),
            pl.BlockSpec((1, w1.shape[0]), lambda i: (0, 0)),
            pl.BlockSpec(w2.shape, lambda i: (0, 0)),
            pl.BlockSpec((1, w2.shape[0]), lambda i: (0, 0)),
            pl.BlockSpec(w3.shape, lambda i: (0, 0)),
            pl.BlockSpec((1, w3.shape[0]), lambda i: (0, 0)),
            pl.BlockSpec(w4.shape, lambda i: (0, 0)),
            pl.BlockSpec((1, w4.shape[0]), lambda i: (0, 0)),
        ],
        out_specs=pl.BlockSpec((nb, w4.shape[0]), lambda i: (0, 0)),
        out_shape=jax.ShapeDtypeStruct((nb, w4.shape[0]), jnp.float32),
        scratch_shapes=[
            pltpu.VMEM((nb, F), jnp.float32),
            pltpu.VMEM((nb, 1), jnp.float32),
        ],
    )(x, bl_row, w1, b1.reshape(1, -1), w2, b2.reshape(1, -1),
      w3, b3.reshape(1, -1), w4, b4.reshape(1, -1))


C = 384
_PAR = pltpu.CompilerParams(dimension_semantics=("parallel",))


def _prep_b_body(st_ref, x_ref, wu_ref, wv_ref, cb_ref, u_ref, v_ref, s_ref):
    g = pl.program_id(0)
    x = x_ref[pl.ds(st_ref[g], C), :]
    u_ref[...] = jax.lax.dot_general(x, wu_ref[...], (((1,), (1,)), ((), ())),
                                     preferred_element_type=jnp.float32) + cb_ref[...]
    v_ref[...] = jax.lax.dot_general(x, wv_ref[...], (((1,), (1,)), ((), ())),
                                     preferred_element_type=jnp.float32)
    sq = jnp.sum(x * x, axis=1, keepdims=True)
    lane = jax.lax.broadcasted_iota(jnp.int32, (C, AUG - F), 1)
    s_ref[:, :F] = x
    s_ref[:, F:] = jnp.where(lane == 0, sq, 0.0)


def _prep_b(x_full, starts, wu, wv, cb):
    nb = starts.shape[0]
    grid_spec = pltpu.PrefetchScalarGridSpec(
        num_scalar_prefetch=1,
        grid=(nb,),
        in_specs=[
            pl.BlockSpec(x_full.shape, lambda i, *_: (0, 0)),
            pl.BlockSpec((F, F), lambda i, *_: (0, 0)),
            pl.BlockSpec((F, F), lambda i, *_: (0, 0)),
            pl.BlockSpec((1, F), lambda i, *_: (0, 0)),
        ],
        out_specs=[
            pl.BlockSpec((C, F), lambda i, *_: (i, 0)),
            pl.BlockSpec((C, F), lambda i, *_: (i, 0)),
            pl.BlockSpec((C, AUG), lambda i, *_: (i, 0)),
        ],
    )
    return pl.pallas_call(
        _prep_b_body,
        grid_spec=grid_spec,
        out_shape=[
            jax.ShapeDtypeStruct((nb * C, F), jnp.float32),
            jax.ShapeDtypeStruct((nb * C, F), jnp.float32),
            jax.ShapeDtypeStruct((nb * C, AUG), jnp.float32),
        ],
        compiler_params=_PAR,
    )(starts, x_full, wu, wv, cb.reshape(1, F))


BP = 16


def _conv_b_body(ls_ref, sd_ref, u_ref, ss_ref, vs_ref, o_ref, w_ref, vmax_ref):
    g = pl.program_id(0)

    lane = jax.lax.broadcasted_iota(jnp.int32, (BP * C, AUG), 1)
    sd = sd_ref[...]
    a = jnp.where(lane < F, -2.0 * sd, jnp.where(lane == F, 1.0, 0.0))
    ciota_c = jax.lax.broadcasted_iota(jnp.int32, (C, C), 1)
    for b in range(BP):
        lsrc = ls_ref[g * BP + b]
        d2 = jax.lax.dot_general(a[b * C:(b + 1) * C],
                                 ss_ref[b * C:(b + 1) * C, :],
                                 (((1,), (1,)), ((), ())),
                                 preferred_element_type=jnp.float32)
        w_ref[b * C:(b + 1) * C, :] = jnp.where(ciota_c < lsrc, d2, jnp.inf)
    vmax_ref[...] = jnp.full((BP * C, F), NEG_INF, jnp.float32)

    ciota = jax.lax.broadcasted_iota(jnp.int32, (BP * C, C), 1)

    def round_body(r, _):
        w = w_ref[...]
        cmin = jnp.min(w, axis=1, keepdims=True)
        lidx = jnp.where(w == cmin, ciota, C)
        ci = jnp.min(lidx, axis=1, keepdims=True)
        active = cmin < jnp.inf
        onehot = (ciota == ci) & active
        w_ref[...] = jnp.where(onehot, jnp.inf, w)
        oh = onehot.astype(jnp.float32)
        for b in range(BP):
            sl = slice(b * C, (b + 1) * C)
            vsel = jax.lax.dot_general(oh[sl], vs_ref[sl, :],
                                       (((1,), (0,)), ((), ())),
                                       preferred_element_type=jnp.float32)
            vmax_ref[sl, :] = jnp.where(active[sl],
                                        jnp.maximum(vmax_ref[sl, :], vsel),
                                        vmax_ref[sl, :])
        return 0

    jax.lax.fori_loop(0, K, round_body, 0, unroll=2)

    vmax = vmax_ref[...]
    res = u_ref[...] + vmax
    o_ref[...] = jnp.where(vmax == NEG_INF, NEG_INF, _elu(res))


def _conv_b(u_dst, s_dst, s_src, v_src, lens_src):
    nb = lens_src.shape[0]
    grid_spec = pltpu.PrefetchScalarGridSpec(
        num_scalar_prefetch=1,
        grid=(nb // BP,),
        in_specs=[
            pl.BlockSpec((BP * C, AUG), lambda i, *_: (i, 0)),
            pl.BlockSpec((BP * C, F), lambda i, *_: (i, 0)),
            pl.BlockSpec((BP * C, AUG), lambda i, *_: (i, 0)),
            pl.BlockSpec((BP * C, F), lambda i, *_: (i, 0)),
        ],
        out_specs=pl.BlockSpec((BP * C, F), lambda i, *_: (i, 0)),
        scratch_shapes=[
            pltpu.VMEM((BP * C, C), jnp.float32),
            pltpu.VMEM((BP * C, F), jnp.float32),
        ],
    )
    return pl.pallas_call(
        _conv_b_body,
        grid_spec=grid_spec,
        out_shape=jax.ShapeDtypeStruct((nb * C, F), jnp.float32),
        compiler_params=_PAR,
    )(lens_src, s_dst, u_dst, s_src, v_src)


def _rounds(w_ref, v_scr, vm_scr, ciota):
    def round_body(r, _):
        w = w_ref[...]
        cmin = jnp.min(w, axis=1, keepdims=True)
        lidx = jnp.where(w == cmin, ciota, C)
        ci = jnp.min(lidx, axis=1, keepdims=True)
        active = cmin < jnp.inf
        onehot = (ciota == ci) & active
        w_ref[...] = jnp.where(onehot, jnp.inf, w)
        oh = onehot.astype(jnp.float32)
        for b in range(BP):
            sl = slice(b * C, (b + 1) * C)
            vsel = jax.lax.dot_general(oh[sl], v_scr[sl, :],
                                       (((1,), (0,)), ((), ())),
                                       preferred_element_type=jnp.float32)
            vm_scr[sl, :] = jnp.where(active[sl],
                                      jnp.maximum(vm_scr[sl, :], vsel),
                                      vm_scr[sl, :])
        return 0

    jax.lax.fori_loop(0, K, round_body, 0, unroll=2)


def _uvw_bucket(xd, xs, lsrc, wu_ref, wv_ref, cb_ref, a_scr, s_scr,
                u_scr, v_scr, w_scr, sl, ciota_c):
    u_scr[sl, :] = jax.lax.dot_general(xd, wu_ref[...], (((1,), (1,)), ((), ())),
                                       preferred_element_type=jnp.float32) + cb_ref[...]
    v_scr[sl, :] = jax.lax.dot_general(xs, wv_ref[...], (((1,), (1,)), ((), ())),
                                       preferred_element_type=jnp.float32)
    sq = jnp.sum(xs * xs, axis=1, keepdims=True)
    lane8 = jax.lax.broadcasted_iota(jnp.int32, (C, AUG - F), 1)
    s_scr[:, :F] = xs
    s_scr[:, F:] = jnp.where(lane8 == 0, sq, 0.0)
    a_scr[:, :F] = -2.0 * xd
    a_scr[:, F:] = jnp.where(lane8 == 0, 1.0, 0.0)
    d2 = jax.lax.dot_general(a_scr[...], s_scr[...], (((1,), (1,)), ((), ())),
                             preferred_element_type=jnp.float32)
    w_scr[sl, :] = jnp.where(ciota_c < lsrc, d2, jnp.inf)


def _chain_body(std_ref, ls_ref, x_ref, w1_ref, b1_ref, w2_ref, b2_ref,
                wu_ref, wv_ref, cb_ref, o_ref,
                w_scr, v_scr, u_scr, f_scr, vm_scr, a_scr, s_scr):
    g = pl.program_id(0)
    ciota_c = jax.lax.broadcasted_iota(jnp.int32, (C, C), 1)
    ciota = jax.lax.broadcasted_iota(jnp.int32, (BP * C, C), 1)

    for b in range(BP):
        st = std_ref[g * BP + b]
        x = x_ref[pl.ds(st, C), :]
        h = _elu(jax.lax.dot_general(x, w1_ref[...], (((1,), (1,)), ((), ())),
                                     preferred_element_type=jnp.float32) + b1_ref[...])
        f_scr[b * C:(b + 1) * C, :] = _elu(
            jax.lax.dot_general(h, w2_ref[...], (((1,), (1,)), ((), ())),
                                preferred_element_type=jnp.float32) + b2_ref[...])

    for _layer in range(2):
        for b in range(BP):
            sl = slice(b * C, (b + 1) * C)
            xc = f_scr[sl, :]
            _uvw_bucket(xc, xc, ls_ref[g * BP + b], wu_ref, wv_ref, cb_ref,
                        a_scr, s_scr, u_scr, v_scr, w_scr, sl, ciota_c)
        vm_scr[...] = jnp.full((BP * C, F), NEG_INF, jnp.float32)
        _rounds(w_scr, v_scr, vm_scr, ciota)
        vmax = vm_scr[...]
        res = u_scr[...] + vmax
        f_scr[...] = jnp.where(vmax == NEG_INF, NEG_INF, _elu(res))

    o_ref[...] = f_scr[...]


def _chain(x_full, starts, lens, w1, b1, w2, b2, wu, wv, cb):
    nb = lens.shape[0]
    fin = x_full.shape[1]
    grid_spec = pltpu.PrefetchScalarGridSpec(
        num_scalar_prefetch=2,
        grid=(nb // BP,),
        in_specs=[
            pl.BlockSpec(x_full.shape, lambda i, *_: (0, 0)),
            pl.BlockSpec((F, fin), lambda i, *_: (0, 0)),
            pl.BlockSpec((1, F), lambda i, *_: (0, 0)),
            pl.BlockSpec((F, F), lambda i, *_: (0, 0)),
            pl.BlockSpec((1, F), lambda i, *_: (0, 0)),
            pl.BlockSpec((F, F), lambda i, *_: (0, 0)),
            pl.BlockSpec((F, F), lambda i, *_: (0, 0)),
            pl.BlockSpec((1, F), lambda i, *_: (0, 0)),
        ],
        out_specs=pl.BlockSpec((BP * C, F), lambda i, *_: (i, 0)),
        scratch_shapes=[
            pltpu.VMEM((BP * C, C), jnp.float32),
            pltpu.VMEM((BP * C, F), jnp.float32),
            pltpu.VMEM((BP * C, F), jnp.float32),
            pltpu.VMEM((BP * C, F), jnp.float32),
            pltpu.VMEM((BP * C, F), jnp.float32),
            pltpu.VMEM((C, AUG), jnp.float32),
            pltpu.VMEM((C, AUG), jnp.float32),
        ],
    )
    return pl.pallas_call(
        _chain_body,
        grid_spec=grid_spec,
        out_shape=jax.ShapeDtypeStruct((nb * C, F), jnp.float32),
        compiler_params=_PAR,
    )(starts, lens, x_full, w1, b1.reshape(1, F), w2, b2.reshape(1, F),
      wu, wv, cb.reshape(1, F))


def _conv5_body(ls_ref, xd_ref, xs_ref, wu_ref, wv_ref, cb_ref, o_ref,
                w_scr, v_scr, u_scr, vm_scr, a_scr, s_scr):
    g = pl.program_id(0)
    ciota_c = jax.lax.broadcasted_iota(jnp.int32, (C, C), 1)
    ciota = jax.lax.broadcasted_iota(jnp.int32, (BP * C, C), 1)
    for b in range(BP):
        sl = slice(b * C, (b + 1) * C)
        _uvw_bucket(xd_ref[sl, :], xs_ref[sl, :], ls_ref[g * BP + b],
                    wu_ref, wv_ref, cb_ref, a_scr, s_scr,
                    u_scr, v_scr, w_scr, sl, ciota_c)
    vm_scr[...] = jnp.full((BP * C, F), NEG_INF, jnp.float32)
    _rounds(w_scr, v_scr, vm_scr, ciota)
    vmax = vm_scr[...]
    res = u_scr[...] + vmax
    o_ref[...] = jnp.where(vmax == NEG_INF, NEG_INF, _elu(res))


def _conv5(xd, xs, lens_src, wu, wv, cb):
    nb = lens_src.shape[0]
    grid_spec = pltpu.PrefetchScalarGridSpec(
        num_scalar_prefetch=1,
        grid=(nb // BP,),
        in_specs=[
            pl.BlockSpec((BP * C, F), lambda i, *_: (i, 0)),
            pl.BlockSpec((BP * C, F), lambda i, *_: (i, 0)),
            pl.BlockSpec((F, F), lambda i, *_: (0, 0)),
            pl.BlockSpec((F, F), lambda i, *_: (0, 0)),
            pl.BlockSpec((1, F), lambda i, *_: (0, 0)),
        ],
        out_specs=pl.BlockSpec((BP * C, F), lambda i, *_: (i, 0)),
        scratch_shapes=[
            pltpu.VMEM((BP * C, C), jnp.float32),
            pltpu.VMEM((BP * C, F), jnp.float32),
            pltpu.VMEM((BP * C, F), jnp.float32),
            pltpu.VMEM((BP * C, F), jnp.float32),
            pltpu.VMEM((C, AUG), jnp.float32),
            pltpu.VMEM((C, AUG), jnp.float32),
        ],
    )
    return pl.pallas_call(
        _conv5_body,
        grid_spec=grid_spec,
        out_shape=jax.ShapeDtypeStruct((nb * C, F), jnp.float32),
        compiler_params=_PAR,
    )(lens_src, xd, xs, wu, wv, cb.reshape(1, F))


def _pool_b_body(ls_ref, x_ref, o_ref):
    g = pl.program_id(0)
    lv = ls_ref[g]
    m = (jax.lax.broadcasted_iota(jnp.int32, (1, C), 1) < lv).astype(jnp.float32)
    s = jax.lax.dot_general(m, x_ref[...], (((1,), (0,)), ((), ())),
                            preferred_element_type=jnp.float32)
    o_ref[...] = (s / jnp.maximum(lv.astype(jnp.float32), 1.0))[None]


def _pool_b(x, lens):
    nb = lens.shape[0]
    grid_spec = pltpu.PrefetchScalarGridSpec(
        num_scalar_prefetch=1,
        grid=(nb,),
        in_specs=[pl.BlockSpec((C, F), lambda i, *_: (i, 0))],
        out_specs=pl.BlockSpec((1, 1, F), lambda i, *_: (i, 0, 0)),
    )
    return pl.pallas_call(
        _pool_b_body,
        grid_spec=grid_spec,
        out_shape=jax.ShapeDtypeStruct((nb, 1, F), jnp.float32),
        compiler_params=_PAR,
    )(lens, x)


def _head_body(p_ref, w1_ref, b1_ref, w2_ref, b2_ref, w3_ref, b3_ref,
               w4_ref, b4_ref, o_ref):
    def lin(h, w_ref, b_ref):
        return jax.lax.dot_general(h, w_ref[...], (((1,), (1,)), ((), ())),
                                   preferred_element_type=jnp.float32) + b_ref[...]

    h = _elu(lin(p_ref[...], w1_ref, b1_ref))
    h = _elu(lin(h, w2_ref, b2_ref))
    h = _elu(lin(h, w3_ref, b3_ref))
    o_ref[...] = lin(h, w4_ref, b4_ref)


def _head(pooled, w1, b1, w2, b2, w3, b3, w4, b4):
    nb = pooled.shape[0]
    return pl.pallas_call(
        _head_body,
        grid=(1,),
        in_specs=[
            pl.BlockSpec((nb, F), lambda i: (0, 0)),
            pl.BlockSpec(w1.shape, lambda i: (0, 0)),
            pl.BlockSpec((1, w1.shape[0]), lambda i: (0, 0)),
            pl.BlockSpec(w2.shape, lambda i: (0, 0)),
            pl.BlockSpec((1, w2.shape[0]), lambda i: (0, 0)),
            pl.BlockSpec(w3.shape, lambda i: (0, 0)),
            pl.BlockSpec((1, w3.shape[0]), lambda i: (0, 0)),
            pl.BlockSpec(w4.shape, lambda i: (0, 0)),
            pl.BlockSpec((1, w4.shape[0]), lambda i: (0, 0)),
        ],
        out_specs=pl.BlockSpec((nb, w4.shape[0]), lambda i: (0, 0)),
        out_shape=jax.ShapeDtypeStruct((nb, w4.shape[0]), jnp.float32),
    )(pooled, w1, b1.reshape(1, -1), w2, b2.reshape(1, -1),
      w3, b3.reshape(1, -1), w4, b4.reshape(1, -1))


def _chunk_info(bd, bs, nb):
    g = jnp.arange(nb, dtype=bs.dtype)
    starts = jnp.searchsorted(bs, g, side="left").astype(jnp.int32)
    ends = jnp.searchsorted(bs, g, side="right").astype(jnp.int32)
    g_lo = bd[::TILE]
    g_hi = bd[TILE - 1::TILE]
    s_lo = starts[g_lo]
    s_hi = ends[g_hi]
    clo = s_lo // CHUNK
    cn = -(-s_hi // CHUNK) - clo
    return clo.astype(jnp.int32), jnp.maximum(cn, 0).astype(jnp.int32)


def kernel(x_ts, x_lc, x_ass, batch_ts, batch_lc, batch_ass,
           ts_w1, ts_b1, ts_w2, ts_b2, lc_w1, lc_b1, lc_w2, lc_b2,
           conv_w, conv_b, out_w1, out_b1, out_w2, out_b2,
           out_w3, out_b3, out_w4, out_b4):
    nb = 32
    n_ts = x_ts.shape[0]
    n_lc = x_lc.shape[0]

    w1c = conv_w[:, :F]
    w2c = conv_w[:, F:]
    wu = w1c - w2c
    wv = w2c

    x_ts_p = jnp.pad(x_ts, ((0, 0), (0, 2)))
    ts_w1_p = jnp.pad(ts_w1, ((0, 0), (0, 2)))
    bts_col = batch_ts.astype(jnp.float32).reshape(n_ts, 1)
    blc_col = batch_lc.astype(jnp.float32).reshape(n_lc, 1)
    bts_row = batch_ts.astype(jnp.float32).reshape(n_ts // CHUNK, CHUNK)
    blc_row = batch_lc.astype(jnp.float32).reshape(n_lc // CHUNK, CHUNK)
    lc_lo, lc_n = _chunk_info(batch_lc, batch_lc, nb)
    ts_lo, ts_n = _chunk_info(batch_ts, batch_ts, nb)
    x_lo, x_n = _chunk_info(batch_lc, batch_ts, nb)

    g = jnp.arange(nb, dtype=jnp.int32)
    st_ts = jnp.searchsorted(batch_ts, g, side="left").astype(jnp.int32)
    en_ts = jnp.searchsorted(batch_ts, g, side="right").astype(jnp.int32)
    st_lc = jnp.searchsorted(batch_lc, g, side="left").astype(jnp.int32)
    en_lc = jnp.searchsorted(batch_lc, g, side="right").astype(jnp.int32)
    len_ts = en_ts - st_ts
    len_lc = en_lc - st_lc
    max_len = jnp.maximum(jnp.max(len_ts), jnp.max(len_lc))
    blk_starts = (g * C).astype(jnp.int32)

    def fast_path(_):
        x_ts_pp = jnp.pad(x_ts_p, ((0, C), (0, 0)))
        x_lc_pp = jnp.pad(x_lc, ((0, C), (0, 0)))
        f11 = _chain(x_lc_pp, st_lc, len_lc, lc_w1, lc_b1, lc_w2, lc_b2,
                     wu, wv, conv_b)
        f22 = _chain(x_ts_pp, st_ts, len_ts, ts_w1_p, ts_b1, ts_w2, ts_b2,
                     wu, wv, conv_b)
        f3 = _conv5(f11, f22, len_ts, wu, wv, conv_b)
        pooled = _pool_b(f3, len_lc).reshape(nb, F)
        return _head(pooled, out_w1, out_b1, out_w2, out_b2,
                     out_w3, out_b3, out_w4, out_b4)

    def slow_path(_):
        enc_ts = _encode(x_ts_p, ts_w1_p, ts_b1, ts_w2, ts_b2)
        enc_lc = _encode(x_lc, lc_w1, lc_b1, lc_w2, lc_b2)

        def conv_same(x, b_col, b_row, clo, cn):
            u, v, s = _prep(x, wu, wv, conv_b)
            return _edge_conv(u, s, b_col, s, v, b_row, clo, cn)

        feats1 = conv_same(enc_lc, blc_col, blc_row, lc_lo, lc_n)
        feats11 = conv_same(feats1, blc_col, blc_row, lc_lo, lc_n)
        feats2 = conv_same(enc_ts, bts_col, bts_row, ts_lo, ts_n)
        feats22 = conv_same(feats2, bts_col, bts_row, ts_lo, ts_n)
        u_d, _, s_d = _prep(feats11, wu, wv, conv_b)
        _, v_s, s_s = _prep(feats22, wu, wv, conv_b)
        feats3 = _edge_conv(u_d, s_d, blc_col, s_s, v_s, bts_row, x_lo, x_n)
        return _pool_head(feats3, blc_row, nb, out_w1, out_b1, out_w2, out_b2,
                          out_w3, out_b3, out_w4, out_b4)

    out = jax.lax.cond(max_len <= C, fast_path, slow_path, 0)
    batch_out = jnp.arange(nb, dtype=jnp.int32)
    return (out, batch_out)


# speedup vs baseline: 51.2788x; 1.0035x over previous
import functools

import jax
import jax.numpy as jnp
from jax.experimental import pallas as pl
from jax.experimental.pallas import tpu as pltpu

F = 128
AUG = 256
TILE = 256
CHUNK = 256
K = 24
NEG_INF = float("-inf")


def _elu(x):
    return jnp.where(x > 0, x, jnp.exp(jnp.minimum(x, 0.0)) - 1.0)


def _encoder_body(x_ref, w1_ref, b1_ref, w2_ref, b2_ref, o_ref):
    h = jax.lax.dot_general(x_ref[...], w1_ref[...], (((1,), (1,)), ((), ())),
                            preferred_element_type=jnp.float32)
    h = _elu(h + b1_ref[...])
    h = jax.lax.dot_general(h, w2_ref[...], (((1,), (1,)), ((), ())),
                            preferred_element_type=jnp.float32)
    o_ref[...] = _elu(h + b2_ref[...])


def _encode(x, w1, b1, w2, b2):
    n, fin = x.shape
    rt = 512
    return pl.pallas_call(
        _encoder_body,
        grid=(n // rt,),
        in_specs=[
            pl.BlockSpec((rt, fin), lambda i: (i, 0)),
            pl.BlockSpec(w1.shape, lambda i: (0, 0)),
            pl.BlockSpec((1, F), lambda i: (0, 0)),
            pl.BlockSpec((F, F), lambda i: (0, 0)),
            pl.BlockSpec((1, F), lambda i: (0, 0)),
        ],
        out_specs=pl.BlockSpec((rt, F), lambda i: (i, 0)),
        out_shape=jax.ShapeDtypeStruct((n, F), jnp.float32),
    )(x, w1, b1.reshape(1, F), w2, b2.reshape(1, F))


def _prep_body(x_ref, wu_ref, wv_ref, cb_ref, u_ref, v_ref, s_ref):
    x = x_ref[...]
    u_ref[...] = jax.lax.dot_general(x, wu_ref[...], (((1,), (1,)), ((), ())),
                                     preferred_element_type=jnp.float32) + cb_ref[...]
    v_ref[...] = jax.lax.dot_general(x, wv_ref[...], (((1,), (1,)), ((), ())),
                                     preferred_element_type=jnp.float32)
    sq = jnp.sum(x * x, axis=1, keepdims=True)
    lane = jax.lax.broadcasted_iota(jnp.int32, (x.shape[0], AUG - F), 1)
    s_ref[:, :F] = x
    s_ref[:, F:] = jnp.where(lane == 0, sq, 0.0)


def _prep(x, wu, wv, cb):
    n = x.shape[0]
    rt = 512
    return pl.pallas_call(
        _prep_body,
        grid=(n // rt,),
        in_specs=[
            pl.BlockSpec((rt, F), lambda i: (i, 0)),
            pl.BlockSpec((F, F), lambda i: (0, 0)),
            pl.BlockSpec((F, F), lambda i: (0, 0)),
            pl.BlockSpec((1, F), lambda i: (0, 0)),
        ],
        out_specs=[
            pl.BlockSpec((rt, F), lambda i: (i, 0)),
            pl.BlockSpec((rt, F), lambda i: (i, 0)),
            pl.BlockSpec((rt, AUG), lambda i: (i, 0)),
        ],
        out_shape=[
            jax.ShapeDtypeStruct((n, F), jnp.float32),
            jax.ShapeDtypeStruct((n, F), jnp.float32),
            jax.ShapeDtypeStruct((n, AUG), jnp.float32),
        ],
    )(x, wu, wv, cb.reshape(1, F))


def _conv_body(clo_ref, cn_ref, sd_ref, u_ref, bd_ref, s_ref, v_ref, bs_ref,
               o_ref, dw_ref, a_ref, vmax_ref, nsrc_chunks):
    t = pl.program_id(0)
    clo = clo_ref[t]
    cn = cn_ref[t]

    lane = jax.lax.broadcasted_iota(jnp.int32, (TILE, AUG), 1)
    sd = sd_ref[...]
    a_ref[...] = jnp.where(lane < F, -2.0 * sd,
                           jnp.where(lane == F, 1.0, 0.0))
    vmax_ref[...] = jnp.full((TILE, F), NEG_INF, jnp.float32)

    bd = bd_ref[...]

    def fill(ql, _):
        q = clo + ql
        sc = s_ref[pl.ds(q * CHUNK, CHUNK), :]
        d2 = jax.lax.dot_general(a_ref[...], sc, (((1,), (1,)), ((), ())),
                                 preferred_element_type=jnp.float32)
        bs = bs_ref[pl.ds(q, 1), :]
        dw_ref[:, pl.ds(ql * CHUNK, CHUNK)] = jnp.where(bd == bs, d2, jnp.inf)
        return 0

    jax.lax.fori_loop(0, cn, fill, 0)

    ciota = jax.lax.broadcasted_iota(jnp.int32, (TILE, CHUNK), 1)

    def round_body(r, _):
        def amin_chunk(ql, carry):
            mv, mi = carry
            w = dw_ref[:, pl.ds(ql * CHUNK, CHUNK)]
            cmin = jnp.min(w, axis=1, keepdims=True)
            lidx = jnp.where(w == cmin, ciota, CHUNK)
            cidx = jnp.min(lidx, axis=1, keepdims=True) + ql * CHUNK
            better = cmin < mv
            return (jnp.where(better, cmin, mv),
                    jnp.where(better, cidx, mi))

        mv0 = jnp.full((TILE, 1), jnp.inf, jnp.float32)
        mi0 = jnp.full((TILE, 1), jnp.int32(-1))
        mv, mi = jax.lax.fori_loop(0, cn, amin_chunk, (mv0, mi0))
        active = mv < jnp.inf

        def sel_chunk(ql, acc):
            base = ql * CHUNK
            onehot = (ciota == (mi - base)) & active
            w = dw_ref[:, pl.ds(base, CHUNK)]
            dw_ref[:, pl.ds(base, CHUNK)] = jnp.where(onehot, jnp.inf, w)
            vchunk = v_ref[pl.ds((clo + ql) * CHUNK, CHUNK), :]
            return acc + jax.lax.dot_general(
                onehot.astype(jnp.float32), vchunk, (((1,), (0,)), ((), ())),
                preferred_element_type=jnp.float32)

        vsel = jax.lax.fori_loop(0, cn, sel_chunk,
                                 jnp.zeros((TILE, F), jnp.float32))
        vmax_ref[...] = jnp.where(active,
                                  jnp.maximum(vmax_ref[...], vsel),
                                  vmax_ref[...])
        return 0

    jax.lax.fori_loop(0, K, round_body, 0)

    vmax = vmax_ref[...]
    res = u_ref[...] + vmax
    o_ref[...] = jnp.where(vmax == NEG_INF, NEG_INF, _elu(res))


def _edge_conv(u_dst, s_dst, bd_col, s_src, v_src, bs_row, clo, cn):
    n_dst = u_dst.shape[0]
    n_src = v_src.shape[0]
    nt = n_dst // TILE
    nsrc_chunks = n_src // CHUNK
    grid_spec = pltpu.PrefetchScalarGridSpec(
        num_scalar_prefetch=2,
        grid=(nt,),
        in_specs=[
            pl.BlockSpec((TILE, AUG), lambda i, *_: (i, 0)),
            pl.BlockSpec((TILE, F), lambda i, *_: (i, 0)),
            pl.BlockSpec((TILE, 1), lambda i, *_: (i, 0)),
            pl.BlockSpec((n_src, AUG), lambda i, *_: (0, 0)),
            pl.BlockSpec((n_src, F), lambda i, *_: (0, 0)),
            pl.BlockSpec((nsrc_chunks, CHUNK), lambda i, *_: (0, 0)),
        ],
        out_specs=pl.BlockSpec((TILE, F), lambda i, *_: (i, 0)),
        scratch_shapes=[
            pltpu.VMEM((TILE, n_src), jnp.float32),
            pltpu.VMEM((TILE, AUG), jnp.float32),
            pltpu.VMEM((TILE, F), jnp.float32),
        ],
    )
    body = functools.partial(_conv_body, nsrc_chunks=nsrc_chunks)
    return pl.pallas_call(
        body,
        grid_spec=grid_spec,
        out_shape=jax.ShapeDtypeStruct((n_dst, F), jnp.float32),
    )(clo, cn, s_dst, u_dst, bd_col, s_src, v_src, bs_row)


def _pool_body(x_ref, bl_ref, w1_ref, b1_ref, w2_ref, b2_ref, w3_ref, b3_ref,
               w4_ref, b4_ref, o_ref, sums_ref, cnt_ref, nb, nt):
    t = pl.program_id(0)

    @pl.when(t == 0)
    def _():
        sums_ref[...] = jnp.zeros_like(sums_ref)
        cnt_ref[...] = jnp.zeros_like(cnt_ref)

    bl = bl_ref[pl.ds(t, 1), :]
    gi = jax.lax.broadcasted_iota(jnp.int32, (nb, TILE), 0)
    oh = (gi == bl.astype(jnp.int32)).astype(jnp.float32)
    sums_ref[...] += jax.lax.dot_general(oh, x_ref[...], (((1,), (0,)), ((), ())),
                                         preferred_element_type=jnp.float32)
    cnt_ref[...] += jnp.sum(oh, axis=1, keepdims=True)

    @pl.when(t == nt - 1)
    def _():
        pooled = sums_ref[...] / jnp.maximum(cnt_ref[...], 1.0)

        def lin(h, w_ref, b_ref):
            return jax.lax.dot_general(h, w_ref[...], (((1,), (1,)), ((), ())),
                                       preferred_element_type=jnp.float32) + b_ref[...]

        h = _elu(lin(pooled, w1_ref, b1_ref))
        h = _elu(lin(h, w2_ref, b2_ref))
        h = _elu(lin(h, w3_ref, b3_ref))
        o_ref[...] = lin(h, w4_ref, b4_ref)


def _pool_head(x, bl_row, nb, w1, b1, w2, b2, w3, b3, w4, b4):
    n = x.shape[0]
    nt = n // TILE
    body = functools.partial(_pool_body, nb=nb, nt=nt)
    return pl.pallas_call(
        body,
        grid=(nt,),
        in_specs=[
            pl.BlockSpec((TILE, F), lambda i: (i, 0)),
            pl.BlockSpec(bl_row.shape, lambda i: (0, 0)),
            pl.BlockSpec(w1.shape, lambda i: (0, 0)),
            pl.BlockSpec((1, w1.shape[0]), lambda i: (0, 0)),
            pl.BlockSpec(w2.shape, lambda i: (0, 0)),
            pl.BlockSpec((1, w2.shape[0]), lambda i: (0, 0)),
            pl.BlockSpec(w3.shape, lambda i: (0, 0)),
            pl.BlockSpec((1, w3.shape[0]), lambda i: (0, 0)),
            pl.BlockSpec(w4.shape, lambda i: (0, 0)),
            pl.BlockSpec((1, w4.shape[0]), lambda i: (0, 0)),
        ],
        out_specs=pl.BlockSpec((nb, w4.shape[0]), lambda i: (0, 0)),
        out_shape=jax.ShapeDtypeStruct((nb, w4.shape[0]), jnp.float32),
        scratch_shapes=[
            pltpu.VMEM((nb, F), jnp.float32),
            pltpu.VMEM((nb, 1), jnp.float32),
        ],
    )(x, bl_row, w1, b1.reshape(1, -1), w2, b2.reshape(1, -1),
      w3, b3.reshape(1, -1), w4, b4.reshape(1, -1))


C = 384
BP = 16
_PAR = pltpu.CompilerParams(dimension_semantics=("parallel",))


def _rounds(w_ref, v_scr, vm_scr, ciota):
    def round_body(r, _):
        w = w_ref[...]
        cmin = jnp.min(w, axis=1, keepdims=True)
        lidx = jnp.where(w == cmin, ciota, C)
        ci = jnp.min(lidx, axis=1, keepdims=True)
        active = cmin < jnp.inf
        onehot = (ciota == ci) & active
        w_ref[...] = jnp.where(onehot, jnp.inf, w)
        oh = onehot.astype(jnp.float32)
        for b in range(BP):
            sl = slice(b * C, (b + 1) * C)
            vsel = jax.lax.dot_general(oh[sl], v_scr[sl, :],
                                       (((1,), (0,)), ((), ())),
                                       preferred_element_type=jnp.float32)
            vm_scr[sl, :] = jnp.where(active[sl],
                                      jnp.maximum(vm_scr[sl, :], vsel),
                                      vm_scr[sl, :])
        return 0

    jax.lax.fori_loop(0, K, round_body, 0)


def _uvw_bucket(xd, xs, lsrc, wu_ref, wv_ref, cb_ref, a_scr, s_scr,
                u_scr, v_scr, w_scr, sl, ciota_c):
    u_scr[sl, :] = jax.lax.dot_general(xd, wu_ref[...], (((1,), (1,)), ((), ())),
                                       preferred_element_type=jnp.float32) + cb_ref[...]
    v_scr[sl, :] = jax.lax.dot_general(xs, wv_ref[...], (((1,), (1,)), ((), ())),
                                       preferred_element_type=jnp.float32)
    sq = jnp.sum(xs * xs, axis=1, keepdims=True)
    lane8 = jax.lax.broadcasted_iota(jnp.int32, (C, AUG - F), 1)
    s_scr[:, :F] = xs
    s_scr[:, F:] = jnp.where(lane8 == 0, sq, 0.0)
    a_scr[:, :F] = -2.0 * xd
    a_scr[:, F:] = jnp.where(lane8 == 0, 1.0, 0.0)
    d2 = jax.lax.dot_general(a_scr[...], s_scr[...], (((1,), (1,)), ((), ())),
                             preferred_element_type=jnp.float32)
    w_scr[sl, :] = jnp.where(ciota_c < lsrc, d2, jnp.inf)


def _chain_body(std_ref, ls_ref, x_ref, w1_ref, b1_ref, w2_ref, b2_ref,
                wu_ref, wv_ref, cb_ref, o_ref,
                w_scr, v_scr, u_scr, f_scr, vm_scr, a_scr, s_scr):
    g = pl.program_id(0)
    ciota_c = jax.lax.broadcasted_iota(jnp.int32, (C, C), 1)
    ciota = jax.lax.broadcasted_iota(jnp.int32, (BP * C, C), 1)

    for b in range(BP):
        st = std_ref[g * BP + b]
        x = x_ref[pl.ds(st, C), :]
        h = _elu(jax.lax.dot_general(x, w1_ref[...], (((1,), (1,)), ((), ())),
                                     preferred_element_type=jnp.float32) + b1_ref[...])
        f_scr[b * C:(b + 1) * C, :] = _elu(
            jax.lax.dot_general(h, w2_ref[...], (((1,), (1,)), ((), ())),
                                preferred_element_type=jnp.float32) + b2_ref[...])

    for _layer in range(2):
        for b in range(BP):
            sl = slice(b * C, (b + 1) * C)
            xc = f_scr[sl, :]
            _uvw_bucket(xc, xc, ls_ref[g * BP + b], wu_ref, wv_ref, cb_ref,
                        a_scr, s_scr, u_scr, v_scr, w_scr, sl, ciota_c)
        vm_scr[...] = jnp.full((BP * C, F), NEG_INF, jnp.float32)
        _rounds(w_scr, v_scr, vm_scr, ciota)
        vmax = vm_scr[...]
        res = u_scr[...] + vmax
        f_scr[...] = jnp.where(vmax == NEG_INF, NEG_INF, _elu(res))

    o_ref[...] = f_scr[...]


def _chain(x_full, starts, lens, w1, b1, w2, b2, wu, wv, cb):
    nb = lens.shape[0]
    fin = x_full.shape[1]
    grid_spec = pltpu.PrefetchScalarGridSpec(
        num_scalar_prefetch=2,
        grid=(nb // BP,),
        in_specs=[
            pl.BlockSpec(x_full.shape, lambda i, *_: (0, 0)),
            pl.BlockSpec((F, fin), lambda i, *_: (0, 0)),
            pl.BlockSpec((1, F), lambda i, *_: (0, 0)),
            pl.BlockSpec((F, F), lambda i, *_: (0, 0)),
            pl.BlockSpec((1, F), lambda i, *_: (0, 0)),
            pl.BlockSpec((F, F), lambda i, *_: (0, 0)),
            pl.BlockSpec((F, F), lambda i, *_: (0, 0)),
            pl.BlockSpec((1, F), lambda i, *_: (0, 0)),
        ],
        out_specs=pl.BlockSpec((BP * C, F), lambda i, *_: (i, 0)),
        scratch_shapes=[
            pltpu.VMEM((BP * C, C), jnp.float32),
            pltpu.VMEM((BP * C, F), jnp.float32),
            pltpu.VMEM((BP * C, F), jnp.float32),
            pltpu.VMEM((BP * C, F), jnp.float32),
            pltpu.VMEM((BP * C, F), jnp.float32),
            pltpu.VMEM((C, AUG), jnp.float32),
            pltpu.VMEM((C, AUG), jnp.float32),
        ],
    )
    return pl.pallas_call(
        _chain_body,
        grid_spec=grid_spec,
        out_shape=jax.ShapeDtypeStruct((nb * C, F), jnp.float32),
        compiler_params=_PAR,
    )(starts, lens, x_full, w1, b1.reshape(1, F), w2, b2.reshape(1, F),
      wu, wv, cb.reshape(1, F))


def _conv5_body(ls_ref, xd_ref, xs_ref, wu_ref, wv_ref, cb_ref, o_ref,
                w_scr, v_scr, u_scr, vm_scr, a_scr, s_scr):
    g = pl.program_id(0)
    ciota_c = jax.lax.broadcasted_iota(jnp.int32, (C, C), 1)
    ciota = jax.lax.broadcasted_iota(jnp.int32, (BP * C, C), 1)
    for b in range(BP):
        sl = slice(b * C, (b + 1) * C)
        _uvw_bucket(xd_ref[sl, :], xs_ref[sl, :], ls_ref[g * BP + b],
                    wu_ref, wv_ref, cb_ref, a_scr, s_scr,
                    u_scr, v_scr, w_scr, sl, ciota_c)
    vm_scr[...] = jnp.full((BP * C, F), NEG_INF, jnp.float32)
    _rounds(w_scr, v_scr, vm_scr, ciota)
    vmax = vm_scr[...]
    res = u_scr[...] + vmax
    o_ref[...] = jnp.where(vmax == NEG_INF, NEG_INF, _elu(res))


def _conv5(xd, xs, lens_src, wu, wv, cb):
    nb = lens_src.shape[0]
    grid_spec = pltpu.PrefetchScalarGridSpec(
        num_scalar_prefetch=1,
        grid=(nb // BP,),
        in_specs=[
            pl.BlockSpec((BP * C, F), lambda i, *_: (i, 0)),
            pl.BlockSpec((BP * C, F), lambda i, *_: (i, 0)),
            pl.BlockSpec((F, F), lambda i, *_: (0, 0)),
            pl.BlockSpec((F, F), lambda i, *_: (0, 0)),
            pl.BlockSpec((1, F), lambda i, *_: (0, 0)),
        ],
        out_specs=pl.BlockSpec((BP * C, F), lambda i, *_: (i, 0)),
        scratch_shapes=[
            pltpu.VMEM((BP * C, C), jnp.float32),
            pltpu.VMEM((BP * C, F), jnp.float32),
            pltpu.VMEM((BP * C, F), jnp.float32),
            pltpu.VMEM((BP * C, F), jnp.float32),
            pltpu.VMEM((C, AUG), jnp.float32),
            pltpu.VMEM((C, AUG), jnp.float32),
        ],
    )
    return pl.pallas_call(
        _conv5_body,
        grid_spec=grid_spec,
        out_shape=jax.ShapeDtypeStruct((nb * C, F), jnp.float32),
        compiler_params=_PAR,
    )(lens_src, xd, xs, wu, wv, cb.reshape(1, F))


def _pool_b_body(ls_ref, x_ref, o_ref):
    g = pl.program_id(0)
    lv = ls_ref[g]
    m = (jax.lax.broadcasted_iota(jnp.int32, (1, C), 1) < lv).astype(jnp.float32)
    s = jax.lax.dot_general(m, x_ref[...], (((1,), (0,)), ((), ())),
                            preferred_element_type=jnp.float32)
    o_ref[...] = (s / jnp.maximum(lv.astype(jnp.float32), 1.0))[None]


def _pool_b(x, lens):
    nb = lens.shape[0]
    grid_spec = pltpu.PrefetchScalarGridSpec(
        num_scalar_prefetch=1,
        grid=(nb,),
        in_specs=[pl.BlockSpec((C, F), lambda i, *_: (i, 0))],
        out_specs=pl.BlockSpec((1, 1, F), lambda i, *_: (i, 0, 0)),
    )
    return pl.pallas_call(
        _pool_b_body,
        grid_spec=grid_spec,
        out_shape=jax.ShapeDtypeStruct((nb, 1, F), jnp.float32),
        compiler_params=_PAR,
    )(lens, x)


def _head_body(p_ref, w1_ref, b1_ref, w2_ref, b2_ref, w3_ref, b3_ref,
               w4_ref, b4_ref, o_ref):
    def lin(h, w_ref, b_ref):
        return jax.lax.dot_general(h, w_ref[...], (((1,), (1,)), ((), ())),
                                   preferred_element_type=jnp.float32) + b_ref[...]

    h = _elu(lin(p_ref[...], w1_ref, b1_ref))
    h = _elu(lin(h, w2_ref, b2_ref))
    h = _elu(lin(h, w3_ref, b3_ref))
    o_ref[...] = lin(h, w4_ref, b4_ref)


def _head(pooled, w1, b1, w2, b2, w3, b3, w4, b4):
    nb = pooled.shape[0]
    return pl.pallas_call(
        _head_body,
        grid=(1,),
        in_specs=[
            pl.BlockSpec((nb, F), lambda i: (0, 0)),
            pl.BlockSpec(w1.shape, lambda i: (0, 0)),
            pl.BlockSpec((1, w1.shape[0]), lambda i: (0, 0)),
            pl.BlockSpec(w2.shape, lambda i: (0, 0)),
            pl.BlockSpec((1, w2.shape[0]), lambda i: (0, 0)),
            pl.BlockSpec(w3.shape, lambda i: (0, 0)),
            pl.BlockSpec((1, w3.shape[0]), lambda i: (0, 0)),
            pl.BlockSpec(w4.shape, lambda i: (0, 0)),
            pl.BlockSpec((1, w4.shape[0]), lambda i: (0, 0)),
        ],
        out_specs=pl.BlockSpec((nb, w4.shape[0]), lambda i: (0, 0)),
        out_shape=jax.ShapeDtypeStruct((nb, w4.shape[0]), jnp.float32),
    )(pooled, w1, b1.reshape(1, -1), w2, b2.reshape(1, -1),
      w3, b3.reshape(1, -1), w4, b4.reshape(1, -1))


def _chunk_info(bd, bs, nb):
    g = jnp.arange(nb, dtype=bs.dtype)
    starts = jnp.searchsorted(bs, g, side="left").astype(jnp.int32)
    ends = jnp.searchsorted(bs, g, side="right").astype(jnp.int32)
    g_lo = bd[::TILE]
    g_hi = bd[TILE - 1::TILE]
    s_lo = starts[g_lo]
    s_hi = ends[g_hi]
    clo = s_lo // CHUNK
    cn = -(-s_hi // CHUNK) - clo
    return clo.astype(jnp.int32), jnp.maximum(cn, 0).astype(jnp.int32)


def kernel(x_ts, x_lc, x_ass, batch_ts, batch_lc, batch_ass,
           ts_w1, ts_b1, ts_w2, ts_b2, lc_w1, lc_b1, lc_w2, lc_b2,
           conv_w, conv_b, out_w1, out_b1, out_w2, out_b2,
           out_w3, out_b3, out_w4, out_b4):
    nb = 32
    n_ts = x_ts.shape[0]
    n_lc = x_lc.shape[0]

    w1c = conv_w[:, :F]
    w2c = conv_w[:, F:]
    wu = w1c - w2c
    wv = w2c

    x_ts_p = jnp.pad(x_ts, ((0, 0), (0, 2)))
    ts_w1_p = jnp.pad(ts_w1, ((0, 0), (0, 2)))
    bts_col = batch_ts.astype(jnp.float32).reshape(n_ts, 1)
    blc_col = batch_lc.astype(jnp.float32).reshape(n_lc, 1)
    bts_row = batch_ts.astype(jnp.float32).reshape(n_ts // CHUNK, CHUNK)
    blc_row = batch_lc.astype(jnp.float32).reshape(n_lc // CHUNK, CHUNK)
    lc_lo, lc_n = _chunk_info(batch_lc, batch_lc, nb)
    ts_lo, ts_n = _chunk_info(batch_ts, batch_ts, nb)
    x_lo, x_n = _chunk_info(batch_lc, batch_ts, nb)

    g = jnp.arange(nb, dtype=jnp.int32)
    st_ts = jnp.searchsorted(batch_ts, g, side="left").astype(jnp.int32)
    en_ts = jnp.searchsorted(batch_ts, g, side="right").astype(jnp.int32)
    st_lc = jnp.searchsorted(batch_lc, g, side="left").astype(jnp.int32)
    en_lc = jnp.searchsorted(batch_lc, g, side="right").astype(jnp.int32)
    len_ts = en_ts - st_ts
    len_lc = en_lc - st_lc
    max_len = jnp.maximum(jnp.max(len_ts), jnp.max(len_lc))

    def fast_path(_):
        x_ts_pp = jnp.pad(x_ts_p, ((0, C), (0, 0)))
        x_lc_pp = jnp.pad(x_lc, ((0, C), (0, 0)))
        f11 = _chain(x_lc_pp, st_lc, len_lc, lc_w1, lc_b1, lc_w2, lc_b2,
                     wu, wv, conv_b)
        f22 = _chain(x_ts_pp, st_ts, len_ts, ts_w1_p, ts_b1, ts_w2, ts_b2,
                     wu, wv, conv_b)
        f3 = _conv5(f11, f22, len_ts, wu, wv, conv_b)
        pooled = _pool_b(f3, len_lc).reshape(nb, F)
        return _head(pooled, out_w1, out_b1, out_w2, out_b2,
                     out_w3, out_b3, out_w4, out_b4)

    def slow_path(_):
        enc_ts = _encode(x_ts_p, ts_w1_p, ts_b1, ts_w2, ts_b2)
        enc_lc = _encode(x_lc, lc_w1, lc_b1, lc_w2, lc_b2)

        def conv_same(x, b_col, b_row, clo, cn):
            u, v, s = _prep(x, wu, wv, conv_b)
            return _edge_conv(u, s, b_col, s, v, b_row, clo, cn)

        feats1 = conv_same(enc_lc, blc_col, blc_row, lc_lo, lc_n)
        feats11 = conv_same(feats1, blc_col, blc_row, lc_lo, lc_n)
        feats2 = conv_same(enc_ts, bts_col, bts_row, ts_lo, ts_n)
        feats22 = conv_same(feats2, bts_col, bts_row, ts_lo, ts_n)
        u_d, _, s_d = _prep(feats11, wu, wv, conv_b)
        _, v_s, s_s = _prep(feats22, wu, wv, conv_b)
        feats3 = _edge_conv(u_d, s_d, blc_col, s_s, v_s, bts_row, x_lo, x_n)
        return _pool_head(feats3, blc_row, nb, out_w1, out_b1, out_w2, out_b2,
                          out_w3, out_b3, out_w4, out_b4)

    out = jax.lax.cond(max_len <= C, fast_path, slow_path, 0)
    batch_out = jnp.arange(nb, dtype=jnp.int32)
    return (out, batch_out)
